# Initial kernel scaffold; baseline (speedup 1.0000x reference)
import jax
import jax.numpy as jnp
from jax.experimental import pallas as pl


def kernel(edge_index, emb, W1, b1, W2, b2):
    raise NotImplementedError("write your pallas kernel here")


# speedup vs baseline: 11.3277x; 11.3277x over previous
import functools

import jax
import jax.numpy as jnp
from jax import lax
from jax.experimental import pallas as pl
from jax.experimental.pallas import tpu as pltpu
from jax.experimental.pallas import tpu_sc as plsc

NC = 2
NS = 16
CHUNK = 128


def _sc_segsum_rows(npad, d, ch, stripe):
  mesh = plsc.VectorSubcoreMesh(core_axis_name="c", subcore_axis_name="s")

  @functools.partial(
      pl.kernel,
      out_type=jax.ShapeDtypeStruct((NC, npad, d), jnp.float32),
      mesh=mesh,
      scratch_types=[
          pltpu.VMEM((ch, CHUNK), jnp.int32),
          pltpu.VMEM((ch, CHUNK), jnp.int32),
          pltpu.VMEM((CHUNK, d), jnp.float32),
          pltpu.SemaphoreType.DMA,
          pltpu.VMEM_SHARED((npad, d), jnp.float32),
      ],
  )
  def k(table, srcw, dstw, zeros, out, idx_s, idx_d, rows, sem, acc):
    c = lax.axis_index("c")
    s = lax.axis_index("s")
    pltpu.sync_copy(srcw.at[c, s], idx_s)
    pltpu.sync_copy(dstw.at[c, s], idx_d)
    pltpu.sync_copy(zeros.at[pl.ds(s * stripe, stripe)],
                    acc.at[pl.ds(s * stripe, stripe)])
    plsc.subcore_barrier()

    def body(j, carry):
      pltpu.async_copy(table.at[idx_s.at[j]], rows, sem).wait()
      pltpu.sync_copy(rows, acc.at[idx_d.at[j]], add=True)
      return carry

    lax.fori_loop(0, ch, body, 0)
    plsc.subcore_barrier()
    pltpu.sync_copy(acc.at[pl.ds(s * stripe, stripe)],
                    out.at[c, pl.ds(s * stripe, stripe)])

  return k


def _sc_degree(npad, d, ch, stripe):
  mesh = plsc.VectorSubcoreMesh(core_axis_name="c", subcore_axis_name="s")

  @functools.partial(
      pl.kernel,
      out_type=jax.ShapeDtypeStruct((NC, npad, d), jnp.float32),
      mesh=mesh,
      scratch_types=[
          pltpu.VMEM((ch, CHUNK), jnp.int32),
          pltpu.VMEM((CHUNK, d), jnp.float32),
          pltpu.VMEM_SHARED((npad, d), jnp.float32),
      ],
  )
  def k(dstw, zeros, ones, out, idx_d, ones_v, acc):
    c = lax.axis_index("c")
    s = lax.axis_index("s")
    pltpu.sync_copy(dstw.at[c, s], idx_d)
    pltpu.sync_copy(ones, ones_v)
    pltpu.sync_copy(zeros.at[pl.ds(s * stripe, stripe)],
                    acc.at[pl.ds(s * stripe, stripe)])
    plsc.subcore_barrier()

    def body(j, carry):
      pltpu.sync_copy(ones_v, acc.at[idx_d.at[j]], add=True)
      return carry

    lax.fori_loop(0, ch, body, 0)
    plsc.subcore_barrier()
    pltpu.sync_copy(acc.at[pl.ds(s * stripe, stripe)],
                    out.at[c, pl.ds(s * stripe, stripe)])

  return k


def _dis_from_degp(degp, n):
  d = degp[0, :, 0:1] + degp[1, :, 0:1] + 1.0
  return lax.rsqrt(d)[:n]


def _tc_layer1(n, npad, d):
  def body(degp_ref, emb_ref, w1_ref, h_ref, g_ref):
    dis = _dis_from_degp(degp_ref[...], n)
    h = jnp.dot(emb_ref[...], w1_ref[...], preferred_element_type=jnp.float32)
    h_ref[...] = h
    g_ref[...] = h * dis

  return pl.pallas_call(
      body,
      out_shape=(jax.ShapeDtypeStruct((n, d), jnp.float32),
                 jax.ShapeDtypeStruct((n, d), jnp.float32)),
  )


def _tc_layer2(n, npad, d):
  def body(s1p_ref, h1_ref, degp_ref, b1_ref, w2_ref, h2_ref, g2_ref):
    dis = _dis_from_degp(degp_ref[...], n)
    s1 = s1p_ref[0, :n, :] + s1p_ref[1, :n, :]
    h1 = h1_ref[...]
    x = jnp.maximum(dis * s1 + h1 * (dis * dis) + b1_ref[...], 0.0)
    h2 = jnp.dot(x, w2_ref[...], preferred_element_type=jnp.float32)
    h2_ref[...] = h2
    g2_ref[...] = h2 * dis

  return pl.pallas_call(
      body,
      out_shape=(jax.ShapeDtypeStruct((n, d), jnp.float32),
                 jax.ShapeDtypeStruct((n, d), jnp.float32)),
  )


def _tc_finish(n, npad, d):
  def body(s2p_ref, h2_ref, degp_ref, b2_ref, out_ref):
    dis = _dis_from_degp(degp_ref[...], n)
    s2 = s2p_ref[0, :n, :] + s2p_ref[1, :n, :]
    o = dis * s2 + h2_ref[...] * (dis * dis) + b2_ref[...]
    row = lax.broadcasted_iota(jnp.int32, (n, 1), 0)
    out_ref[...] = jnp.where(row == 0, 0.0, o)

  return pl.pallas_call(
      body,
      out_shape=jax.ShapeDtypeStruct((n, d), jnp.float32),
  )


def kernel(edge_index, emb, W1, b1, W2, b2):
  n, d = emb.shape
  e = edge_index.shape[1]
  nw = NC * NS
  ch = -(-e // (nw * CHUNK))
  e_pad = nw * ch * CHUNK
  npad = -(-(n + 1) // 128) * 128
  stripe = npad // NS

  src = edge_index[0]
  dst = edge_index[1]
  pad = e_pad - e
  srcw = jnp.concatenate([src, jnp.zeros((pad,), jnp.int32)]).reshape(
      NC, NS, ch, CHUNK)
  dstw = jnp.concatenate([dst, jnp.full((pad,), n, jnp.int32)]).reshape(
      NC, NS, ch, CHUNK)

  zeros_nd = jnp.zeros((npad, d), jnp.float32)
  ones_dg = jnp.ones((CHUNK, d), jnp.float32)

  degp = _sc_degree(npad, d, ch, stripe)(dstw, zeros_nd, ones_dg)

  h1, g1 = _tc_layer1(n, npad, d)(degp, emb, W1)
  segsum = _sc_segsum_rows(npad, d, ch, stripe)
  s1p = segsum(g1, srcw, dstw, zeros_nd)

  h2, g2 = _tc_layer2(n, npad, d)(
      s1p, h1, degp, b1.reshape(1, d), W2)
  s2p = segsum(g2, srcw, dstw, zeros_nd)

  return _tc_finish(n, npad, d)(s2p, h2, degp, b2.reshape(1, d))
